# Initial kernel scaffold; baseline (speedup 1.0000x reference)
import jax
import jax.numpy as jnp
from jax.experimental import pallas as pl


def kernel(x, W1s, W2s, Wr, W1e, W2e):
    raise NotImplementedError("write your pallas kernel here")


# speedup vs baseline: 2.7997x; 2.7997x over previous
import functools

import jax
import jax.numpy as jnp
from jax.experimental import pallas as pl
from jax.experimental.pallas import tpu as pltpu

_N_EMBD = 768
_N_EXP = 64
_E_DIM = 192
_N_TOK = 8192
_TB = 256
_B = 128
_NB_MAX = _N_TOK // _B + _N_EXP


def _router_body(x_ref, wr_ref, eid_ref):
    logits = jnp.dot(x_ref[...], wr_ref[...], preferred_element_type=jnp.float32)
    m = jnp.max(logits, axis=1, keepdims=True)
    col = jax.lax.broadcasted_iota(jnp.int32, logits.shape, 1)
    eid = jnp.min(jnp.where(logits == m, col, _N_EXP), axis=1)
    eid_ref[0, 0, :] = eid


def _router(x, Wr):
    nb = _N_TOK // _TB
    eid3 = pl.pallas_call(
        _router_body,
        grid=(nb,),
        in_specs=[
            pl.BlockSpec((_TB, _N_EMBD), lambda i: (i, 0)),
            pl.BlockSpec((_N_EMBD, _N_EXP), lambda i: (0, 0)),
        ],
        out_specs=pl.BlockSpec((1, 1, _TB), lambda i: (i, 0, 0)),
        out_shape=jax.ShapeDtypeStruct((nb, 1, _TB), jnp.int32),
    )(x, Wr)
    return eid3.reshape(_N_TOK)


def _ffn_body(be_ref, xs_ref, w1s_ref, w2s_ref, w1e_ref, w2e_ref, out_ref):
    xb = xs_ref[...]
    hs = jnp.dot(xb, w1s_ref[...], preferred_element_type=jnp.float32)
    hs = jnp.square(jnp.maximum(hs, 0.0))
    acc = jnp.dot(hs, w2s_ref[...], preferred_element_type=jnp.float32)
    he = jnp.dot(xb, w1e_ref[0], preferred_element_type=jnp.float32)
    he = jnp.square(jnp.maximum(he, 0.0))
    acc = acc + jnp.dot(he, w2e_ref[0], preferred_element_type=jnp.float32)
    out_ref[...] = acc


def _grouped_ffn(xs_pad, W1s, W2s, W1e, W2e, block_expert):
    grid_spec = pltpu.PrefetchScalarGridSpec(
        num_scalar_prefetch=1,
        grid=(_NB_MAX,),
        in_specs=[
            pl.BlockSpec((_B, _N_EMBD), lambda b, be: (b, 0)),
            pl.BlockSpec((_N_EMBD, _E_DIM), lambda b, be: (0, 0)),
            pl.BlockSpec((_E_DIM, _N_EMBD), lambda b, be: (0, 0)),
            pl.BlockSpec((1, _N_EMBD, _E_DIM), lambda b, be: (be[b], 0, 0)),
            pl.BlockSpec((1, _E_DIM, _N_EMBD), lambda b, be: (be[b], 0, 0)),
        ],
        out_specs=pl.BlockSpec((_B, _N_EMBD), lambda b, be: (b, 0)),
    )
    return pl.pallas_call(
        _ffn_body,
        grid_spec=grid_spec,
        out_shape=jax.ShapeDtypeStruct((_NB_MAX * _B, _N_EMBD), jnp.float32),
    )(block_expert, xs_pad, W1s, W2s, W1e, W2e)


def _dispatch(eid):
    counts = jnp.bincount(eid, length=_N_EXP)
    sort_idx = jnp.argsort(eid)
    tiles = (counts + _B - 1) // _B
    cum_tiles = jnp.cumsum(tiles)
    blk = jnp.arange(_NB_MAX, dtype=jnp.int32)
    be = jnp.searchsorted(cum_tiles, blk, side="right").astype(jnp.int32)
    be = jnp.minimum(be, _N_EXP - 1)
    tile_start = cum_tiles - tiles
    exp_start = jnp.cumsum(counts) - counts
    rank = (blk - tile_start[be])[:, None] * _B + jnp.arange(_B)[None, :]
    valid = rank < counts[be][:, None]
    pos = exp_start[be][:, None] + rank
    padded_idx = jnp.where(valid, sort_idx[jnp.clip(pos, 0, _N_TOK - 1)], 0)
    padded_idx = padded_idx.reshape(-1).astype(jnp.int32)
    flat_valid = valid.reshape(-1)
    inv = jnp.zeros((_N_TOK,), jnp.int32).at[
        jnp.where(flat_valid, padded_idx, _N_TOK)
    ].set(jnp.arange(_NB_MAX * _B, dtype=jnp.int32), mode="drop")
    return be, padded_idx, inv


@jax.jit
def kernel(x, W1s, W2s, Wr, W1e, W2e):
    x_flat = x.reshape(-1, _N_EMBD)
    eid = _router(x_flat, Wr)
    be, padded_idx, inv = _dispatch(eid)
    xs_pad = jnp.take(x_flat, padded_idx, axis=0)
    out_pad = _grouped_ffn(xs_pad, W1s, W2s, W1e, W2e, be)
    out = jnp.take(out_pad, inv, axis=0)
    return out.reshape(x.shape)
